# baseline (speedup 1.0000x reference)
import jax
import jax.numpy as jnp
from jax.experimental import pallas as pl
from jax.experimental.pallas import tpu as pltpu

_NBUF = 4
_BB = 64


def _se1d_pipeline_kernel(x_hbm, w1_ref, b1_ref, w2_ref, b2_ref, o_hbm,
                          xbuf, obuf, insem, outsem):
    bsz = x_hbm.shape[0]
    length = x_hbm.shape[2]
    steps = bsz // _BB
    inv_len = 1.0 / length

    def in_copy(i, slot):
        return pltpu.make_async_copy(
            x_hbm.at[pl.ds(i * _BB, _BB)], xbuf.at[slot], insem.at[slot])

    def out_copy(i, slot):
        return pltpu.make_async_copy(
            obuf.at[slot], o_hbm.at[pl.ds(i * _BB, _BB)], outsem.at[slot])

    for d in range(_NBUF):
        in_copy(d, d).start()

    def body(i, carry):
        slot = jax.lax.rem(i, _NBUF)

        @pl.when(i >= _NBUF)
        def _():
            out_copy(i - _NBUF, slot).wait()

        in_copy(i, slot).wait()

        xf = xbuf[slot]
        y = jnp.sum(xf, axis=-1) * inv_len
        z = jnp.maximum(
            jnp.dot(y, w1_ref[...], preferred_element_type=jnp.float32)
            + b1_ref[...], 0.0)
        s = jax.nn.sigmoid(
            jnp.dot(z, w2_ref[...], preferred_element_type=jnp.float32)
            + b2_ref[...])
        obuf[slot] = xf * s[:, :, None]

        out_copy(i, slot).start(priority=1)

        @pl.when(i + _NBUF < steps)
        def _():
            in_copy(i + _NBUF, slot).start()

        return carry

    jax.lax.fori_loop(0, steps, body, 0, unroll=False)

    for d in range(_NBUF):
        i = steps - _NBUF + d
        out_copy(i, jax.lax.rem(i, _NBUF)).wait()


@jax.jit
def _se_layer_1d(x, fc1_w, fc1_b, fc2_w, fc2_b):
    bsz, c, length = x.shape
    bott = fc1_w.shape[0]

    w1_t = jnp.transpose(fc1_w).astype(jnp.float32)
    w2_t = jnp.transpose(fc2_w).astype(jnp.float32)
    b1 = fc1_b.reshape(1, bott).astype(jnp.float32)
    b2 = fc2_b.reshape(1, c).astype(jnp.float32)

    lanes = max(length, 128)
    buf_bytes = 2 * _NBUF * _BB * c * lanes * 4
    cost = pl.CostEstimate(
        flops=int(4 * bsz * c * bott + 2 * bsz * c * length),
        transcendentals=int(bsz * c),
        bytes_accessed=int(2 * bsz * c * length * 4
                           + 4 * (2 * c * bott + bott + c)),
    )
    return pl.pallas_call(
        _se1d_pipeline_kernel,
        out_shape=jax.ShapeDtypeStruct((bsz, c, length), x.dtype),
        in_specs=[
            pl.BlockSpec(memory_space=pl.ANY),
            pl.BlockSpec(memory_space=pltpu.MemorySpace.VMEM),
            pl.BlockSpec(memory_space=pltpu.MemorySpace.VMEM),
            pl.BlockSpec(memory_space=pltpu.MemorySpace.VMEM),
            pl.BlockSpec(memory_space=pltpu.MemorySpace.VMEM),
        ],
        out_specs=pl.BlockSpec(memory_space=pl.ANY),
        scratch_shapes=[
            pltpu.VMEM((_NBUF, _BB, c, length), jnp.float32),
            pltpu.VMEM((_NBUF, _BB, c, length), jnp.float32),
            pltpu.SemaphoreType.DMA((_NBUF,)),
            pltpu.SemaphoreType.DMA((_NBUF,)),
        ],
        compiler_params=pltpu.CompilerParams(
            vmem_limit_bytes=int(min(buf_bytes + (8 << 20), 56 << 20)),
        ),
        cost_estimate=cost,
    )(x, w1_t, b1, w2_t, b2)


def kernel(x, fc1_w, fc1_b, fc2_w, fc2_b):
    return _se_layer_1d(x, fc1_w, fc1_b, fc2_w, fc2_b)
